# baseline (speedup 1.0000x reference)
import functools

import jax
import jax.numpy as jnp
from jax import lax
from jax.experimental import pallas as pl
from jax.experimental.pallas import tpu as pltpu
from jax.experimental.pallas import tpu_sc as plsc


_N = 32 * 1 * 1024 * 256
_NW = 32
_PER_W = _N // _NW
_CHUNK = 16384
_NCHUNK = _PER_W // _CHUNK

_LN2 = 0.6931471805599453
_C5 = 0.030102625011692218
_C4 = -0.2806325404497544
_C3 = 1.1048082361995168
_C2 = -2.420812563219248
_C1 = 3.498227901209959
_C0 = -1.9316715417209647 - 127.0 * _LN2


def _masked_log_acc(h, s, acc, cnt):
    bits = lax.bitcast_convert_type(s, jnp.int32)
    e = (bits >> 23).astype(jnp.float32)
    m = lax.bitcast_convert_type((bits & 0x7FFFFF) | 0x3F800000, jnp.float32)
    p = _C5
    for c in (_C4, _C3, _C2, _C1, _C0):
        p = p * m + c
    t = e * _LN2 + p
    return acc + h * t, cnt + h


def _sc_loss_body(h_hbm, s_hbm, part_out, h_v, s_v, stage_v):
    wid = lax.axis_index("s") * 2 + lax.axis_index("c")
    base = wid * _PER_W

    def chunk_body(ci, carry):
        acc, cnt = carry
        off = base + ci * _CHUNK
        pltpu.sync_copy(h_hbm.at[pl.ds(off, _CHUNK)], h_v)
        pltpu.sync_copy(s_hbm.at[pl.ds(off, _CHUNK)], s_v)

        def reg_body(i, carry2):
            acc2, cnt2 = carry2
            for u in range(4):
                sl = pl.ds((i * 4 + u) * 16, 16)
                acc2, cnt2 = _masked_log_acc(h_v[sl], s_v[sl], acc2, cnt2)
            return acc2, cnt2

        return lax.fori_loop(0, _CHUNK // 64, reg_body, (acc, cnt))

    zero = jnp.zeros((16,), jnp.float32)
    acc, cnt = lax.fori_loop(0, _NCHUNK, chunk_body, (zero, zero))
    for j in range(8):
        stage_v[pl.ds(j * 16, 16)] = zero
    stage_v[pl.ds(0, 16)] = acc
    stage_v[pl.ds(16, 16)] = cnt
    pltpu.sync_copy(stage_v, part_out.at[pl.ds(wid * 128, 128)])


_sc_loss = functools.partial(
    pl.kernel,
    out_type=jax.ShapeDtypeStruct((_NW * 128,), jnp.float32),
    mesh=plsc.VectorSubcoreMesh(core_axis_name="c", subcore_axis_name="s"),
    scratch_types=[
        pltpu.VMEM((_CHUNK,), jnp.float32),
        pltpu.VMEM((_CHUNK,), jnp.float32),
        pltpu.VMEM((128,), jnp.float32),
    ],
)(_sc_loss_body)


def kernel(hard_attention, soft_attention):
    h1 = hard_attention.reshape(_N)
    s1 = soft_attention.reshape(_N)
    parts = _sc_loss(h1, s1).reshape(_NW, 128)
    return -jnp.sum(parts[:, :16]) / jnp.sum(parts[:, 16:32])


# speedup vs baseline: 2.2384x; 2.2384x over previous
import functools

import jax
import jax.numpy as jnp
from jax import lax
from jax.experimental import pallas as pl
from jax.experimental.pallas import tpu as pltpu
from jax.experimental.pallas import tpu_sc as plsc


_ROWS = 32 * 1024
_COLS = 256
_NW = 32
_CHUNK_ROWS = 64

_TC_ROWS = 0
_W_ROWS = (_ROWS - _TC_ROWS) // _NW
_NCHUNK = _W_ROWS // _CHUNK_ROWS

_LN2 = 0.6931471805599453
_C4 = -0.05486285286208111
_C3 = 0.4358618497761762
_C2 = -1.4424810126031888
_C1 = 2.7922552255841686
_C0 = -1.7306316977196963 - 127.0 * _LN2


def _masked_log_acc(h, s, acc, cnt):
    bits = lax.bitcast_convert_type(s, jnp.int32)
    e = (bits >> 23).astype(jnp.float32)
    m = lax.bitcast_convert_type((bits & 0x7FFFFF) | 0x3F800000, jnp.float32)
    p = _C4
    for c in (_C3, _C2, _C1, _C0):
        p = p * m + c
    t = e * _LN2 + p
    return acc + h * t, cnt + h


def _sc_loss_body(h_hbm, s_hbm, part_out, h_v, s_v, stage_v,
                  hsem0, ssem0, hsem1, ssem1):
    wid = lax.axis_index("s") * 2 + lax.axis_index("c")
    row0 = _TC_ROWS + wid * _W_ROWS
    sems = ((hsem0, ssem0), (hsem1, ssem1))

    def copies(ci, b):
        off = row0 + ci * _CHUNK_ROWS
        hs, ss = sems[b]
        return (
            pltpu.make_async_copy(
                h_hbm.at[pl.ds(off, _CHUNK_ROWS)], h_v.at[b], hs),
            pltpu.make_async_copy(
                s_hbm.at[pl.ds(off, _CHUNK_ROWS)], s_v.at[b], ss),
        )

    for b in range(2):
        for c in copies(b, b):
            c.start()

    def chunk_pair(ci2, carry):
        acc, cnt = carry
        for b in range(2):
            ci = ci2 * 2 + b
            for c in copies(ci, b):
                c.wait()

            def rows(r, carry2):
                acc2, cnt2 = carry2
                for c in range(_COLS // 16):
                    sl = pl.ds(c * 16, 16)
                    acc2, cnt2 = _masked_log_acc(
                        h_v[b, r, sl], s_v[b, r, sl], acc2, cnt2)
                return acc2, cnt2

            acc, cnt = lax.fori_loop(0, _CHUNK_ROWS, rows, (acc, cnt))

            @pl.when(ci2 < _NCHUNK // 2 - 1)
            def _prefetch():
                for c in copies(ci + 2, b):
                    c.start()
        return acc, cnt

    zero = jnp.zeros((16,), jnp.float32)
    acc, cnt = lax.fori_loop(0, _NCHUNK // 2, chunk_pair, (zero, zero))

    for j in range(8):
        stage_v[pl.ds(j * 16, 16)] = zero
    stage_v[pl.ds(0, 16)] = acc
    stage_v[pl.ds(16, 16)] = cnt
    pltpu.sync_copy(stage_v, part_out.at[pl.ds(wid * 128, 128)])


_sc_loss = functools.partial(
    pl.kernel,
    out_type=jax.ShapeDtypeStruct((_NW * 128,), jnp.float32),
    mesh=plsc.VectorSubcoreMesh(core_axis_name="c", subcore_axis_name="s"),
    compiler_params=pltpu.CompilerParams(use_tc_tiling_on_sc=True),
    scratch_types=[
        pltpu.VMEM((2, _CHUNK_ROWS, _COLS), jnp.float32),
        pltpu.VMEM((2, _CHUNK_ROWS, _COLS), jnp.float32),
        pltpu.VMEM((128,), jnp.float32),
        pltpu.SemaphoreType.DMA,
        pltpu.SemaphoreType.DMA,
        pltpu.SemaphoreType.DMA,
        pltpu.SemaphoreType.DMA,
    ],
)(_sc_loss_body)


def kernel(hard_attention, soft_attention):
    h2 = hard_attention.reshape(_ROWS, _COLS)
    s2 = soft_attention.reshape(_ROWS, _COLS)
    parts = _sc_loss(h2, s2).reshape(_NW, 128)
    return -jnp.sum(parts[:, :16]) / jnp.sum(parts[:, 16:32])


# speedup vs baseline: 3.9447x; 1.7623x over previous
import functools

import jax
import jax.numpy as jnp
from jax import lax
from jax.experimental import pallas as pl
from jax.experimental.pallas import tpu as pltpu
from jax.experimental.pallas import tpu_sc as plsc


_ROWS = 32 * 1024
_COLS = 256
_NW = 32
_CHUNK_ROWS = 64

_TC_ROWS = 24576
_TC_BLOCK_ROWS = 4096
_W_ROWS = (_ROWS - _TC_ROWS) // _NW
_NCHUNK = _W_ROWS // _CHUNK_ROWS

_LN2 = 0.6931471805599453
_C4 = -0.05486285286208111
_C3 = 0.4358618497761762
_C2 = -1.4424810126031888
_C1 = 2.7922552255841686
_C0 = -1.7306316977196963 - 127.0 * _LN2


def _masked_log_acc(h, s, acc, cnt):
    bits = lax.bitcast_convert_type(s, jnp.int32)
    e = (bits >> 23).astype(jnp.float32)
    m = lax.bitcast_convert_type((bits & 0x7FFFFF) | 0x3F800000, jnp.float32)
    p = _C4
    for c in (_C3, _C2, _C1, _C0):
        p = p * m + c
    t = e * _LN2 + p
    return acc + h * t, cnt + h


def _sc_loss_body(h_hbm, s_hbm, part_out, h_v, s_v, stage_v,
                  hsem0, ssem0, hsem1, ssem1):
    wid = lax.axis_index("s") * 2 + lax.axis_index("c")
    row0 = _TC_ROWS + wid * _W_ROWS
    sems = ((hsem0, ssem0), (hsem1, ssem1))

    def copies(ci, b):
        off = row0 + ci * _CHUNK_ROWS
        hs, ss = sems[b]
        return (
            pltpu.make_async_copy(
                h_hbm.at[pl.ds(off, _CHUNK_ROWS)], h_v.at[b], hs),
            pltpu.make_async_copy(
                s_hbm.at[pl.ds(off, _CHUNK_ROWS)], s_v.at[b], ss),
        )

    for b in range(2):
        for c in copies(b, b):
            c.start()

    def chunk_pair(ci2, carry):
        acc, cnt = carry
        for b in range(2):
            ci = ci2 * 2 + b
            for c in copies(ci, b):
                c.wait()

            def rows(r, carry2):
                acc2, cnt2 = carry2
                for c in range(_COLS // 16):
                    sl = pl.ds(c * 16, 16)
                    acc2, cnt2 = _masked_log_acc(
                        h_v[b, r, sl], s_v[b, r, sl], acc2, cnt2)
                return acc2, cnt2

            acc, cnt = lax.fori_loop(0, _CHUNK_ROWS, rows, (acc, cnt))

            @pl.when(ci2 < _NCHUNK // 2 - 1)
            def _prefetch():
                for c in copies(ci + 2, b):
                    c.start()
        return acc, cnt

    zero = jnp.zeros((16,), jnp.float32)
    acc, cnt = lax.fori_loop(0, _NCHUNK // 2, chunk_pair, (zero, zero))

    for j in range(8):
        stage_v[pl.ds(j * 16, 16)] = zero
    stage_v[pl.ds(0, 16)] = acc
    stage_v[pl.ds(16, 16)] = cnt
    pltpu.sync_copy(stage_v, part_out.at[pl.ds(wid * 128, 128)])


_sc_loss = functools.partial(
    pl.kernel,
    out_type=jax.ShapeDtypeStruct((_NW * 128,), jnp.float32),
    mesh=plsc.VectorSubcoreMesh(core_axis_name="c", subcore_axis_name="s"),
    compiler_params=pltpu.CompilerParams(use_tc_tiling_on_sc=True),
    scratch_types=[
        pltpu.VMEM((2, _CHUNK_ROWS, _COLS), jnp.float32),
        pltpu.VMEM((2, _CHUNK_ROWS, _COLS), jnp.float32),
        pltpu.VMEM((128,), jnp.float32),
        pltpu.SemaphoreType.DMA,
        pltpu.SemaphoreType.DMA,
        pltpu.SemaphoreType.DMA,
        pltpu.SemaphoreType.DMA,
    ],
)(_sc_loss_body)


def _tc_loss_body(hard_ref, soft_ref, logsum_ref, count_ref):
    i = pl.program_id(0)

    @pl.when(i == 0)
    def _init():
        logsum_ref[...] = jnp.zeros_like(logsum_ref)
        count_ref[...] = jnp.zeros_like(count_ref)

    h = hard_ref[...]
    s = soft_ref[...]
    logsum_ref[...] += jnp.sum(h * jnp.log(s)).reshape(1, 1)
    count_ref[...] += jnp.sum(h).reshape(1, 1)


def _tc_loss(h2, s2):
    return pl.pallas_call(
        _tc_loss_body,
        grid=(_TC_ROWS // _TC_BLOCK_ROWS,),
        in_specs=[
            pl.BlockSpec((_TC_BLOCK_ROWS, _COLS), lambda i: (i, 0)),
            pl.BlockSpec((_TC_BLOCK_ROWS, _COLS), lambda i: (i, 0)),
        ],
        out_specs=[
            pl.BlockSpec((1, 1), lambda i: (0, 0)),
            pl.BlockSpec((1, 1), lambda i: (0, 0)),
        ],
        out_shape=[
            jax.ShapeDtypeStruct((1, 1), jnp.float32),
            jax.ShapeDtypeStruct((1, 1), jnp.float32),
        ],
    )(h2, s2)


def kernel(hard_attention, soft_attention):
    h2 = hard_attention.reshape(_ROWS, _COLS)
    s2 = soft_attention.reshape(_ROWS, _COLS)
    parts = _sc_loss(h2, s2).reshape(_NW, 128)
    tc_log, tc_cnt = _tc_loss(h2, s2)
    logsum = jnp.sum(parts[:, :16]) + tc_log[0, 0]
    count = jnp.sum(parts[:, 16:32]) + tc_cnt[0, 0]
    return -logsum / count


# speedup vs baseline: 4.2306x; 1.0725x over previous
import functools

import jax
import jax.numpy as jnp
from jax import lax
from jax.experimental import pallas as pl
from jax.experimental.pallas import tpu as pltpu
from jax.experimental.pallas import tpu_sc as plsc


_ROWS = 32 * 1024
_COLS = 256
_NW = 32
_CHUNK_ROWS = 64

_TC_ROWS = 28672
_TC_BLOCK_ROWS = 4096
_W_ROWS = (_ROWS - _TC_ROWS) // _NW
_NCHUNK = _W_ROWS // _CHUNK_ROWS

_LN2 = 0.6931471805599453
_C4 = -0.05486285286208111
_C3 = 0.4358618497761762
_C2 = -1.4424810126031888
_C1 = 2.7922552255841686
_C0 = -1.7306316977196963 - 127.0 * _LN2


def _masked_log_acc(h, s, acc, cnt):
    bits = lax.bitcast_convert_type(s, jnp.int32)
    e = (bits >> 23).astype(jnp.float32)
    m = lax.bitcast_convert_type((bits & 0x7FFFFF) | 0x3F800000, jnp.float32)
    p = _C4
    for c in (_C3, _C2, _C1, _C0):
        p = p * m + c
    t = e * _LN2 + p
    return acc + h * t, cnt + h


def _sc_loss_body(h_hbm, s_hbm, part_out, h_v, s_v, stage_v,
                  hsem0, ssem0, hsem1, ssem1):
    wid = lax.axis_index("s") * 2 + lax.axis_index("c")
    row0 = _TC_ROWS + wid * _W_ROWS
    sems = ((hsem0, ssem0), (hsem1, ssem1))

    def copies(ci, b):
        off = row0 + ci * _CHUNK_ROWS
        hs, ss = sems[b]
        return (
            pltpu.make_async_copy(
                h_hbm.at[pl.ds(off, _CHUNK_ROWS)], h_v.at[b], hs),
            pltpu.make_async_copy(
                s_hbm.at[pl.ds(off, _CHUNK_ROWS)], s_v.at[b], ss),
        )

    for b in range(2):
        for c in copies(b, b):
            c.start()

    def chunk_pair(ci2, carry):
        acc, cnt = carry
        for b in range(2):
            ci = ci2 * 2 + b
            for c in copies(ci, b):
                c.wait()

            def rows(r, carry2):
                acc2, cnt2 = carry2
                for c in range(_COLS // 16):
                    sl = pl.ds(c * 16, 16)
                    acc2, cnt2 = _masked_log_acc(
                        h_v[b, r, sl], s_v[b, r, sl], acc2, cnt2)
                return acc2, cnt2

            acc, cnt = lax.fori_loop(0, _CHUNK_ROWS, rows, (acc, cnt))

            @pl.when(ci2 < _NCHUNK // 2 - 1)
            def _prefetch():
                for c in copies(ci + 2, b):
                    c.start()
        return acc, cnt

    zero = jnp.zeros((16,), jnp.float32)
    acc, cnt = lax.fori_loop(0, _NCHUNK // 2, chunk_pair, (zero, zero))

    for j in range(8):
        stage_v[pl.ds(j * 16, 16)] = zero
    stage_v[pl.ds(0, 16)] = acc
    stage_v[pl.ds(16, 16)] = cnt
    pltpu.sync_copy(stage_v, part_out.at[pl.ds(wid * 128, 128)])


_sc_loss = functools.partial(
    pl.kernel,
    out_type=jax.ShapeDtypeStruct((_NW * 128,), jnp.float32),
    mesh=plsc.VectorSubcoreMesh(core_axis_name="c", subcore_axis_name="s"),
    compiler_params=pltpu.CompilerParams(
        use_tc_tiling_on_sc=True, skip_device_barrier=True),
    scratch_types=[
        pltpu.VMEM((2, _CHUNK_ROWS, _COLS), jnp.float32),
        pltpu.VMEM((2, _CHUNK_ROWS, _COLS), jnp.float32),
        pltpu.VMEM((128,), jnp.float32),
        pltpu.SemaphoreType.DMA,
        pltpu.SemaphoreType.DMA,
        pltpu.SemaphoreType.DMA,
        pltpu.SemaphoreType.DMA,
    ],
)(_sc_loss_body)


def _tc_loss_body(hard_ref, soft_ref, logsum_ref, count_ref):
    i = pl.program_id(0)

    @pl.when(i == 0)
    def _init():
        logsum_ref[...] = jnp.zeros_like(logsum_ref)
        count_ref[...] = jnp.zeros_like(count_ref)

    h = hard_ref[...]
    s = soft_ref[...]
    logsum_ref[...] += jnp.sum(h * jnp.log(s)).reshape(1, 1)
    count_ref[...] += jnp.sum(h).reshape(1, 1)


def _tc_loss(h2, s2):
    return pl.pallas_call(
        _tc_loss_body,
        grid=(_TC_ROWS // _TC_BLOCK_ROWS,),
        in_specs=[
            pl.BlockSpec((_TC_BLOCK_ROWS, _COLS), lambda i: (i, 0)),
            pl.BlockSpec((_TC_BLOCK_ROWS, _COLS), lambda i: (i, 0)),
        ],
        out_specs=[
            pl.BlockSpec((1, 1), lambda i: (0, 0)),
            pl.BlockSpec((1, 1), lambda i: (0, 0)),
        ],
        out_shape=[
            jax.ShapeDtypeStruct((1, 1), jnp.float32),
            jax.ShapeDtypeStruct((1, 1), jnp.float32),
        ],
    )(h2, s2)


def kernel(hard_attention, soft_attention):
    h2 = hard_attention.reshape(_ROWS, _COLS)
    s2 = soft_attention.reshape(_ROWS, _COLS)
    parts = _sc_loss(h2, s2).reshape(_NW, 128)
    tc_log, tc_cnt = _tc_loss(h2, s2)
    logsum = jnp.sum(parts[:, :16]) + tc_log[0, 0]
    count = jnp.sum(parts[:, 16:32]) + tc_cnt[0, 0]
    return -logsum / count


# speedup vs baseline: 7.5760x; 1.7908x over previous
import jax
import jax.numpy as jnp
from jax.experimental import pallas as pl


_ROWS = 32 * 1 * 1024
_COLS = 256
_BLOCK_ROWS = 4096
_GRID = _ROWS // _BLOCK_ROWS


def _loss_body(hard_ref, soft_ref, logsum_ref, count_ref):
    i = pl.program_id(0)

    @pl.when(i == 0)
    def _init():
        logsum_ref[...] = jnp.zeros_like(logsum_ref)
        count_ref[...] = jnp.zeros_like(count_ref)

    h = hard_ref[...]
    s = soft_ref[...]
    logsum_ref[...] += jnp.sum(h * jnp.log(s)).reshape(1, 1)
    count_ref[...] += jnp.sum(h).reshape(1, 1)


def kernel(hard_attention, soft_attention):
    h2 = hard_attention.reshape(_ROWS, _COLS)
    s2 = soft_attention.reshape(_ROWS, _COLS)
    logsum, count = pl.pallas_call(
        _loss_body,
        grid=(_GRID,),
        in_specs=[
            pl.BlockSpec((_BLOCK_ROWS, _COLS), lambda i: (i, 0)),
            pl.BlockSpec((_BLOCK_ROWS, _COLS), lambda i: (i, 0)),
        ],
        out_specs=[
            pl.BlockSpec((1, 1), lambda i: (0, 0)),
            pl.BlockSpec((1, 1), lambda i: (0, 0)),
        ],
        out_shape=[
            jax.ShapeDtypeStruct((1, 1), jnp.float32),
            jax.ShapeDtypeStruct((1, 1), jnp.float32),
        ],
    )(h2, s2)
    return -logsum[0, 0] / count[0, 0]
